# baseline (speedup 1.0000x reference)
import jax
import jax.numpy as jnp
from jax.experimental import pallas as pl

_B = 2048
_C = 512
_R = 64
_K = _C * _R
_NOUT = 8
_BT = 128


def _red_kernel(x_ref, we_ref, wo_ref, o_ref):
    xb = x_ref[...]
    a = jnp.sum(xb[:, :, :_R], axis=2)
    b = jnp.sum(xb[:, :, _R:], axis=2)
    dn = (((1,), (0,)), ((), ()))
    o_ref[...] = (
        jax.lax.dot_general(a, we_ref[...], dn, preferred_element_type=jnp.float32)
        + jax.lax.dot_general(b, wo_ref[...], dn, preferred_element_type=jnp.float32)
    )


def kernel(x, start_boundaries, W_loc, b_loc, W_conf, b_conf):
    del start_boundaries
    x3 = x.reshape(_B, _C // 2, 2 * _R)
    Wc = jnp.concatenate([W_loc, W_conf], axis=0)
    Wc = jnp.pad(Wc, ((0, _NOUT - 5), (0, 0)))
    w = Wc.T / _R
    w_even = w[0::2, :]
    w_odd = w[1::2, :]

    out = pl.pallas_call(
        _red_kernel,
        grid=(_B // _BT,),
        in_specs=[
            pl.BlockSpec((_BT, _C // 2, 2 * _R), lambda i: (i, 0, 0)),
            pl.BlockSpec((_C // 2, _NOUT), lambda i: (0, 0)),
            pl.BlockSpec((_C // 2, _NOUT), lambda i: (0, 0)),
        ],
        out_specs=pl.BlockSpec((_BT, _NOUT), lambda i: (i, 0)),
        out_shape=jax.ShapeDtypeStruct((_B, _NOUT), jnp.float32),
    )(x3, w_even, w_odd)

    loc = out[:, :2] + b_loc
    conf = out[:, 2:5] + b_conf
    return (loc, conf)


# speedup vs baseline: 11.3001x; 11.3001x over previous
import jax
import jax.numpy as jnp
from jax.experimental import pallas as pl

_B = 2048
_C = 512
_T = 16
_H = 2
_NOUT = 8
_BT = 64


def _pool_kernel(x_ref, w_ref, o_ref):
    xb = x_ref[...]
    s = jnp.sum(xb, axis=(1, 2))
    acc = jnp.zeros((xb.shape[0], _NOUT), jnp.float32)
    for u in range(8):
        acc = acc + jax.lax.dot_general(
            s[:, u, :], w_ref[u],
            (((1,), (0,)), ((), ())),
            preferred_element_type=jnp.float32,
        )
    o_ref[...] = acc


def kernel(x, start_boundaries, W_loc, b_loc, W_conf, b_conf):
    del start_boundaries
    x6 = x.reshape(_B, 4, 128, _T, _H, 2)
    xv = x6.transpose(0, 3, 4, 1, 5, 2).reshape(_B, _T, _H, 8, 128)

    Wc = jnp.concatenate([W_loc, W_conf], axis=0)
    Wc = jnp.pad(Wc, ((0, _NOUT - 5), (0, 0)))
    wv4 = (Wc.T / (_T * _H * 2)).reshape(4, 1, 128, _NOUT)
    wv = jnp.tile(wv4, (1, 2, 1, 1)).reshape(8, 128, _NOUT)

    out = pl.pallas_call(
        _pool_kernel,
        grid=(_B // _BT,),
        in_specs=[
            pl.BlockSpec((_BT, _T, _H, 8, 128), lambda i: (i, 0, 0, 0, 0)),
            pl.BlockSpec((8, 128, _NOUT), lambda i: (0, 0, 0)),
        ],
        out_specs=pl.BlockSpec((_BT, _NOUT), lambda i: (i, 0)),
        out_shape=jax.ShapeDtypeStruct((_B, _NOUT), jnp.float32),
    )(xv, wv)

    loc = out[:, :2] + b_loc
    conf = out[:, 2:5] + b_conf
    return (loc, conf)
